# baseline (speedup 1.0000x reference)
import jax
import jax.numpy as jnp
from jax.experimental import pallas as pl
from jax.experimental.pallas import tpu as pltpu


_ROWS = 256


def _dist_body(xf_ref, xc_ref, y_ref,
               amin_ref, srf_ref, cmf_ref, src_ref, cmc_ref):
    y = y_ref[0]
    m = y.shape[0]
    y2 = jnp.sum(y * y, axis=1, keepdims=True).T

    def one_cloud(x_ref, sr_ref, cm_ref, want_amin):
        x = x_ref[0]
        x2 = jnp.sum(x * x, axis=1, keepdims=True)
        ab = jax.lax.dot_general(x, y, (((1,), (1,)), ((), ())),
                                 preferred_element_type=jnp.float32)
        d = jnp.maximum((x2 + y2) - 2.0 * ab, 0.0)
        rowmin = jnp.min(d, axis=1, keepdims=True)
        if want_amin:
            lane = jax.lax.broadcasted_iota(jnp.int32, d.shape, 1)
            amin_ref[0] = jnp.min(jnp.where(d == rowmin, lane, m), axis=1,
                                  keepdims=True)
        srow = jnp.sum(jnp.sqrt(rowmin))
        sr_ref[...] = srow[None, None, None, None]
        cm_ref[0, 0] = jnp.min(d, axis=0, keepdims=True)

    one_cloud(xf_ref, srf_ref, cmf_ref, True)
    one_cloud(xc_ref, src_ref, cmc_ref, False)


def _triple(f):
    px, py, pz = f[:, 0, :], f[:, 1, :], f[:, 2, :]
    qx = jnp.roll(px, -1, axis=1)
    qy = jnp.roll(py, -1, axis=1)
    qz = jnp.roll(pz, -1, axis=1)
    rx = jnp.roll(px, -2, axis=1)
    ry = jnp.roll(py, -2, axis=1)
    rz = jnp.roll(pz, -2, axis=1)
    return (px * (qy * rz - qz * ry)
            + py * (qz * rx - qx * rz)
            + pz * (qx * ry - qy * rx))


def _loss_body(fT_ref, ntT_ref, srf_ref, cmf_ref, src_ref, cmc_ref,
               cham_ref, sqrefl_ref, fz2_ref, gz2_ref, tf_ref, tg_ref):
    f = fT_ref[...]
    g = ntT_ref[...]
    bn = f.shape[0] * f.shape[2]

    def chamfer(sr_ref, cm_ref):
        colmin = jnp.min(cm_ref[...], axis=1)
        scol = jnp.sum(jnp.sqrt(colmin))
        return (jnp.sum(sr_ref[...]) / bn + scol / bn) / 2.0

    cham = chamfer(srf_ref, cmf_ref) + chamfer(src_ref, cmc_ref)
    cham_ref[...] = cham[None, None]
    def proj(v):
        return v.astype(jnp.bfloat16).astype(jnp.float32)

    fy = proj(f[:, 1, :])
    gy = proj(g[:, 1, :])
    sqrefl_ref[...] = (fy - gy) ** 2
    fz = proj(f[:, 2, :])
    gz = proj(g[:, 2, :])
    fz2_ref[...] = fz * fz
    gz2_ref[...] = gz * gz
    tf_ref[...] = _triple(f)
    tg_ref[...] = _triple(g)


def kernel(source_points, target_points):
    coarse = source_points[0]
    fine = source_points[1]
    B, N, _ = fine.shape
    M = target_points.shape[1]
    R = _ROWS
    T = N // R
    fineT = jnp.swapaxes(fine, 1, 2)

    pts_spec = pl.BlockSpec((1, R, 3), lambda b, t: (b, t, 0))
    tgt_spec = pl.BlockSpec((1, M, 3), lambda b, t: (b, 0, 0))
    amin_spec = pl.BlockSpec((1, R, 1), lambda b, t: (b, t, 0))
    sr_spec = pl.BlockSpec((1, 1, 1, 1), lambda b, t: (b, t, 0, 0))
    cm_spec = pl.BlockSpec((1, 1, 1, M), lambda b, t: (b, t, 0, 0))

    amin, srf, cmf, src, cmc = pl.pallas_call(
        _dist_body,
        grid=(B, T),
        in_specs=[pts_spec, pts_spec, tgt_spec],
        out_specs=[amin_spec, sr_spec, cm_spec, sr_spec, cm_spec],
        out_shape=[
            jax.ShapeDtypeStruct((B, N, 1), jnp.int32),
            jax.ShapeDtypeStruct((B, T, 1, 1), jnp.float32),
            jax.ShapeDtypeStruct((B, T, 1, M), jnp.float32),
            jax.ShapeDtypeStruct((B, T, 1, 1), jnp.float32),
            jax.ShapeDtypeStruct((B, T, 1, M), jnp.float32),
        ],
        compiler_params=pltpu.CompilerParams(
            dimension_semantics=("arbitrary", "arbitrary")),
    )(fine, coarse, target_points)

    nt = jnp.take_along_axis(target_points, amin, axis=1)
    ntT = jnp.swapaxes(nt, 1, 2)

    bn_shape = jax.ShapeDtypeStruct((B, N), jnp.float32)
    cham, sqrefl, fz2, gz2, tf, tg = pl.pallas_call(
        _loss_body,
        in_specs=[pl.BlockSpec(a.shape, lambda n=a.ndim: (0,) * n)
                  for a in (fineT, ntT, srf, cmf, src, cmc)],
        out_specs=[pl.BlockSpec((1, 1), lambda: (0, 0))]
        + [pl.BlockSpec((B, N), lambda: (0, 0))] * 5,
        out_shape=[jax.ShapeDtypeStruct((1, 1), jnp.float32)] + [bn_shape] * 5,
    )(fineT, ntT, srf, cmf, src, cmc)

    in_mag = jnp.sqrt(jnp.sum(fz2, axis=-1))
    re_mag = jnp.sqrt(jnp.sum(gz2, axis=-1))
    loss_rot = jnp.mean((in_mag - re_mag) ** 2)
    loss_refl = jnp.mean(sqrefl)
    vol_f = jnp.sum(tf, axis=1) / 6.0
    vol_g = jnp.sum(tg, axis=1) / 6.0
    loss_geo = jnp.mean((vol_f - vol_g) ** 2)
    return loss_rot + loss_refl + cham[0, 0] + loss_geo


# speedup vs baseline: 1.3087x; 1.3087x over previous
import jax
import jax.numpy as jnp
from jax.experimental import pallas as pl
from jax.experimental.pallas import tpu as pltpu


_ROWS = 256


def _dist_body(xf_ref, xc_ref, y_ref, y2_ref,
               amin_ref, srf_ref, cmf_ref, src_ref, cmc_ref):
    y = y_ref[0]
    m = y.shape[0]
    y2 = y2_ref[0]

    def one_cloud(x_ref, sr_ref, cm_ref, want_amin):
        x = x_ref[0]
        x2 = jnp.sum(x * x, axis=1, keepdims=True)
        ab = jax.lax.dot_general(x, y, (((1,), (1,)), ((), ())),
                                 preferred_element_type=jnp.float32)
        d = jnp.maximum((x2 + y2) - 2.0 * ab, 0.0)
        rowmin = jnp.min(d, axis=1, keepdims=True)
        if want_amin:
            lane = jax.lax.broadcasted_iota(jnp.int32, d.shape, 1)
            amin_ref[0] = jnp.min(jnp.where(d == rowmin, lane, m), axis=1,
                                  keepdims=True)
        srow = jnp.sum(jnp.sqrt(rowmin))
        sr_ref[...] = srow[None, None, None, None]
        cm_ref[0, 0] = jnp.min(d, axis=0, keepdims=True)

    one_cloud(xf_ref, srf_ref, cmf_ref, True)
    one_cloud(xc_ref, src_ref, cmc_ref, False)


def _triple(f):
    px, py, pz = f[:, 0, :], f[:, 1, :], f[:, 2, :]
    qx = jnp.roll(px, -1, axis=1)
    qy = jnp.roll(py, -1, axis=1)
    qz = jnp.roll(pz, -1, axis=1)
    rx = jnp.roll(px, -2, axis=1)
    ry = jnp.roll(py, -2, axis=1)
    rz = jnp.roll(pz, -2, axis=1)
    return (px * (qy * rz - qz * ry)
            + py * (qz * rx - qx * rz)
            + pz * (qx * ry - qy * rx))


def _loss_body(fT_ref, ntT_ref, srf_ref, cmf_ref, src_ref, cmc_ref,
               cham_ref, sqrefl_ref, fz2_ref, gz2_ref, tf_ref, tg_ref):
    f = fT_ref[...]
    g = ntT_ref[...]
    bn = f.shape[0] * f.shape[2]

    def chamfer(sr_ref, cm_ref):
        colmin = jnp.min(cm_ref[...], axis=1)
        scol = jnp.sum(jnp.sqrt(colmin))
        return (jnp.sum(sr_ref[...]) / bn + scol / bn) / 2.0

    cham = chamfer(srf_ref, cmf_ref) + chamfer(src_ref, cmc_ref)
    cham_ref[...] = cham[None, None]
    def proj(v):
        return v.astype(jnp.bfloat16).astype(jnp.float32)

    fy = proj(f[:, 1, :])
    gy = proj(g[:, 1, :])
    sqrefl_ref[...] = (fy - gy) ** 2
    fz = proj(f[:, 2, :])
    gz = proj(g[:, 2, :])
    fz2_ref[...] = fz * fz
    gz2_ref[...] = gz * gz
    tf_ref[...] = _triple(f)
    tg_ref[...] = _triple(g)


def kernel(source_points, target_points):
    coarse = source_points[0]
    fine = source_points[1]
    B, N, _ = fine.shape
    M = target_points.shape[1]
    R = _ROWS
    T = N // R
    fineT = jnp.swapaxes(fine, 1, 2)

    y2 = jnp.sum(target_points * target_points, axis=-1)[:, None, :]

    pts_spec = pl.BlockSpec((1, R, 3), lambda b, t: (b, t, 0))
    tgt_spec = pl.BlockSpec((1, M, 3), lambda b, t: (b, 0, 0))
    y2_spec = pl.BlockSpec((1, 1, M), lambda b, t: (b, 0, 0))
    amin_spec = pl.BlockSpec((1, R, 1), lambda b, t: (b, t, 0))
    sr_spec = pl.BlockSpec((1, 1, 1, 1), lambda b, t: (b, t, 0, 0))
    cm_spec = pl.BlockSpec((1, 1, 1, M), lambda b, t: (b, t, 0, 0))

    amin, srf, cmf, src, cmc = pl.pallas_call(
        _dist_body,
        grid=(B, T),
        in_specs=[pts_spec, pts_spec, tgt_spec, y2_spec],
        out_specs=[amin_spec, sr_spec, cm_spec, sr_spec, cm_spec],
        out_shape=[
            jax.ShapeDtypeStruct((B, N, 1), jnp.int32),
            jax.ShapeDtypeStruct((B, T, 1, 1), jnp.float32),
            jax.ShapeDtypeStruct((B, T, 1, M), jnp.float32),
            jax.ShapeDtypeStruct((B, T, 1, 1), jnp.float32),
            jax.ShapeDtypeStruct((B, T, 1, M), jnp.float32),
        ],
        compiler_params=pltpu.CompilerParams(
            dimension_semantics=("arbitrary", "arbitrary")),
    )(fine, coarse, target_points, y2)

    nt = jnp.take_along_axis(target_points, amin, axis=1)
    ntT = jnp.swapaxes(nt, 1, 2)

    bn_shape = jax.ShapeDtypeStruct((B, N), jnp.float32)
    cham, sqrefl, fz2, gz2, tf, tg = pl.pallas_call(
        _loss_body,
        in_specs=[pl.BlockSpec(a.shape, lambda n=a.ndim: (0,) * n)
                  for a in (fineT, ntT, srf, cmf, src, cmc)],
        out_specs=[pl.BlockSpec((1, 1), lambda: (0, 0))]
        + [pl.BlockSpec((B, N), lambda: (0, 0))] * 5,
        out_shape=[jax.ShapeDtypeStruct((1, 1), jnp.float32)] + [bn_shape] * 5,
    )(fineT, ntT, srf, cmf, src, cmc)

    in_mag = jnp.sqrt(jnp.sum(fz2, axis=-1))
    re_mag = jnp.sqrt(jnp.sum(gz2, axis=-1))
    loss_rot = jnp.mean((in_mag - re_mag) ** 2)
    loss_refl = jnp.mean(sqrefl)
    vol_f = jnp.sum(tf, axis=1) / 6.0
    vol_g = jnp.sum(tg, axis=1) / 6.0
    loss_geo = jnp.mean((vol_f - vol_g) ** 2)
    return loss_rot + loss_refl + cham[0, 0] + loss_geo
